# baseline (speedup 1.0000x reference)
import functools

import jax
import jax.numpy as jnp
from jax import lax
from jax.experimental import pallas as pl
from jax.experimental.pallas import tpu as pltpu
from jax.experimental.pallas import tpu_sc as plsc

VOCAB = 1000000
D = 64
A = 4096
S = 200
B = A * S

_info = plsc.get_sparse_core_info()
NC, NS = _info.num_cores, _info.num_subcores
NW = NC * NS

SB = 256
NFULL = VOCAB // SB
TAIL = VOCAB - NFULL * SB
NPAIR = (NFULL // NW) // 2

UNITS = (S * (A // 128))
UPC = UNITS // NW


def _iota16():
    return lax.iota(jnp.int32, 16)


@functools.partial(
    pl.kernel,
    mesh=plsc.VectorSubcoreMesh(core_axis_name="c", subcore_axis_name="s"),
    out_type=jax.ShapeDtypeStruct((VOCAB // 2, 128), jnp.float32),
    compiler_params=pltpu.CompilerParams(needs_layout_passes=False),
    scratch_types=[
        pltpu.VMEM((64, SB), jnp.float32),
        pltpu.VMEM((64, SB), jnp.float32),
        pltpu.VMEM((128, 128), jnp.float32),
        pltpu.VMEM((128, 128), jnp.float32),
        pltpu.SemaphoreType.DMA,
        pltpu.SemaphoreType.DMA,
    ],
)
def _convert_kernel(tt_hbm, tail_hbm, tc_hbm, tin0, tin1, ro0, ro1, r0, r1):
    wid = lax.axis_index("s") * NC + lax.axis_index("c")
    iota = _iota16()

    def fire_reads(u, tin, sem):
        for ct in range(8):
            pltpu.async_copy(
                tt_hbm.at[pl.ds(8 * ct, 8), pl.ds(u * SB, SB)],
                tin.at[pl.ds(8 * ct, 8), :], sem)

    def wait_reads(u, tin, sem):
        for ct in range(8):
            pltpu.make_async_copy(
                tt_hbm.at[pl.ds(8 * ct, 8), pl.ds(u * SB, SB)],
                tin.at[pl.ds(8 * ct, 8), :], sem).wait()

    def transpose(tin, ro, nrow):
        def body(p, carry):
            for m in range(8):
                lq = 2 * p + (m // 4)
                c0 = 16 * (m % 4)
                vals = plsc.load_gather(
                    tin, [c0 + iota, jnp.full((16,), 0, jnp.int32) + lq])
                ro[p, pl.ds(16 * m, 16)] = vals
            return carry
        lax.fori_loop(0, nrow, body, None)

    def full_block(u, tin, ro, sem):
        wait_reads(u, tin, sem)
        transpose(tin, ro, 128)
        pltpu.sync_copy(ro, tc_hbm.at[pl.ds(u * 128, 128), :])

    fire_reads(wid, tin0, r0)

    def pair(g, carry):
        u0 = wid + NW * (2 * g)
        u1 = wid + NW * (2 * g + 1)
        fire_reads(u1, tin1, r1)
        full_block(u0, tin0, ro0, r0)

        @pl.when(jnp.logical_or(g < NPAIR - 1, wid < 2))
        def _():
            fire_reads(wid + NW * (2 * g + 2), tin0, r0)
        full_block(u1, tin1, ro1, r1)
        return carry

    lax.fori_loop(0, NPAIR, pair, None)

    @pl.when(wid < 2)
    def _():
        full_block(wid + NW * (2 * NPAIR), tin0, ro0, r0)

    @pl.when(wid == 2)
    def _():
        pltpu.sync_copy(tail_hbm, ro0.at[pl.ds(0, TAIL // 2), :])
        pltpu.sync_copy(ro0.at[pl.ds(0, TAIL // 2), :],
                        tc_hbm.at[pl.ds(NFULL * 128, TAIL // 2), :])


@functools.partial(
    pl.kernel,
    mesh=plsc.VectorSubcoreMesh(core_axis_name="c", subcore_axis_name="s"),
    out_type=jax.ShapeDtypeStruct((S, D, A), jnp.float32),
    compiler_params=pltpu.CompilerParams(
        use_tc_tiling_on_sc=False, needs_layout_passes=False),
    scratch_types=[
        pltpu.VMEM((128,), jnp.int32),
        pltpu.VMEM((128,), jnp.int32),
        pltpu.VMEM((128, D), jnp.float32),
        pltpu.VMEM((128, D), jnp.float32),
        pltpu.VMEM((D, 128), jnp.float32),
        pltpu.VMEM((D, 128), jnp.float32),
        pltpu.SemaphoreType.DMA,
        pltpu.SemaphoreType.DMA,
        pltpu.SemaphoreType.DMA,
        pltpu.SemaphoreType.DMA,
    ],
)
def _gather_kernel(xt_hbm, tbl_hbm, out_hbm,
                   idx0, idx1, rows0, rows1, ot0, ot1, g0, g1, w0, w1):
    wid = lax.axis_index("s") * NC + lax.axis_index("c")
    base = wid * UPC
    iota = _iota16()

    def out_slice(k):
        u = base + k
        b = u // (A // 128)
        t = u % (A // 128)
        return out_hbm.at[b, :, pl.ds(t * 128, 128)]

    def load_idx(k, idxv):
        pltpu.sync_copy(xt_hbm.at[pl.ds((base + k) * 128, 128)], idxv)

    def fire_gather(idxv, rows, sem):
        pltpu.async_copy(tbl_hbm.at[idxv], rows, sem)

    def step(k, idxv, rows, ot, gsem, wsem):
        pltpu.make_async_copy(tbl_hbm.at[idxv], rows, gsem).wait()

        @pl.when(k >= 2)
        def _():
            pltpu.make_async_copy(ot, out_slice(k - 2), wsem).wait()

        def body(c, carry):
            for lb in range(8):
                vals = plsc.load_gather(
                    rows, [16 * lb + iota, jnp.full((16,), 0, jnp.int32) + c])
                ot[c, pl.ds(16 * lb, 16)] = vals
            return carry
        lax.fori_loop(0, D, body, None)

        pltpu.async_copy(ot, out_slice(k), wsem)

        @pl.when(k + 2 < UPC)
        def _():
            load_idx(k + 2, idxv)
            fire_gather(idxv, rows, gsem)

    load_idx(0, idx0)
    fire_gather(idx0, rows0, g0)
    load_idx(1, idx1)
    fire_gather(idx1, rows1, g1)

    def pair(g, carry):
        step(2 * g, idx0, rows0, ot0, g0, w0)
        step(2 * g + 1, idx1, rows1, ot1, g1, w1)
        return carry

    lax.fori_loop(0, UPC // 2, pair, None)

    pltpu.make_async_copy(ot0, out_slice(UPC - 2), w0).wait()
    pltpu.make_async_copy(ot1, out_slice(UPC - 1), w1).wait()


def kernel(x, table):
    tail = table[NFULL * SB:].reshape(TAIL // 2, 128)
    table_c = _convert_kernel(table.T, tail)
    xt = x.T.reshape(-1)
    out_p = _gather_kernel(xt, table_c.reshape(VOCAB, D))
    return out_p.transpose(2, 0, 1)


# speedup vs baseline: 1.0599x; 1.0599x over previous
import functools

import jax
import jax.numpy as jnp
from jax import lax
from jax.experimental import pallas as pl
from jax.experimental.pallas import tpu as pltpu
from jax.experimental.pallas import tpu_sc as plsc

VOCAB = 1000000
D = 64
A = 4096
S = 200
B = A * S

_info = plsc.get_sparse_core_info()
NC, NS = _info.num_cores, _info.num_subcores
NW = NC * NS

SB = 256
NFULL = VOCAB // SB
TAIL = VOCAB - NFULL * SB
NPAIR = (NFULL // NW) // 2

UNITS = (S * (A // 128))
UPC = UNITS // NW


def _iota16():
    return lax.iota(jnp.int32, 16)


@functools.partial(
    pl.kernel,
    mesh=plsc.VectorSubcoreMesh(core_axis_name="c", subcore_axis_name="s"),
    out_type=jax.ShapeDtypeStruct((VOCAB // 2, 128), jnp.float32),
    compiler_params=pltpu.CompilerParams(needs_layout_passes=False),
    scratch_types=[
        pltpu.VMEM((64, SB), jnp.float32),
        pltpu.VMEM((64, SB), jnp.float32),
        pltpu.VMEM((128, 128), jnp.float32),
        pltpu.VMEM((128, 128), jnp.float32),
        pltpu.SemaphoreType.DMA,
        pltpu.SemaphoreType.DMA,
        pltpu.SemaphoreType.DMA,
        pltpu.SemaphoreType.DMA,
    ],
)
def _convert_kernel(tt_hbm, tail_hbm, tc_hbm, tin0, tin1, ro0, ro1,
                    r0, r1, w0, w1):
    wid = lax.axis_index("s") * NC + lax.axis_index("c")
    iota = _iota16()

    def fire_reads(u, tin, sem):
        for ct in range(8):
            pltpu.async_copy(
                tt_hbm.at[pl.ds(8 * ct, 8), pl.ds(u * SB, SB)],
                tin.at[pl.ds(8 * ct, 8), :], sem)

    def wait_reads(u, tin, sem):
        for ct in range(8):
            pltpu.make_async_copy(
                tt_hbm.at[pl.ds(8 * ct, 8), pl.ds(u * SB, SB)],
                tin.at[pl.ds(8 * ct, 8), :], sem).wait()

    tbase = [16 * (m % 4) + iota for m in range(8)]

    def transpose(tin, ro, nrow):
        def body(p, carry):
            v0 = jnp.full((16,), 0, jnp.int32) + 2 * p
            v1 = v0 + 1
            for m in range(8):
                lvec = v0 if m < 4 else v1
                vals = plsc.load_gather(tin, [tbase[m], lvec])
                ro[p, pl.ds(16 * m, 16)] = vals
            return carry
        lax.fori_loop(0, nrow, body, None)

    def wdesc(u, ro, wsem):
        return pltpu.make_async_copy(
            ro, tc_hbm.at[pl.ds(u * 128, 128), :], wsem)

    def full_block(g, u, tin, ro, rsem, wsem):
        wait_reads(u, tin, rsem)

        @pl.when(g > 0)
        def _():
            wdesc(u - 2 * NW, ro, wsem).wait()
        transpose(tin, ro, 128)
        pltpu.async_copy(ro, tc_hbm.at[pl.ds(u * 128, 128), :], wsem)

    fire_reads(wid, tin0, r0)

    def pair(g, carry):
        u0 = wid + NW * (2 * g)
        u1 = wid + NW * (2 * g + 1)
        fire_reads(u1, tin1, r1)
        full_block(g, u0, tin0, ro0, r0, w0)

        @pl.when(jnp.logical_or(g < NPAIR - 1, wid < 2))
        def _():
            fire_reads(wid + NW * (2 * g + 2), tin0, r0)
        full_block(g, u1, tin1, ro1, r1, w1)
        return carry

    lax.fori_loop(0, NPAIR, pair, None)

    @pl.when(wid < 2)
    def _():
        u = wid + NW * (2 * NPAIR)
        wait_reads(u, tin0, r0)
        wdesc(u - 2 * NW, ro0, w0).wait()
        transpose(tin0, ro0, 128)
        pltpu.async_copy(ro0, tc_hbm.at[pl.ds(u * 128, 128), :], w0)
        wdesc(u, ro0, w0).wait()

    @pl.when(wid >= 2)
    def _():
        wdesc(wid + NW * (2 * NPAIR - 2), ro0, w0).wait()
    wdesc(wid + NW * (2 * NPAIR - 1), ro1, w1).wait()

    @pl.when(wid == 2)
    def _():
        pltpu.sync_copy(tail_hbm, ro0.at[pl.ds(0, TAIL // 2), :])
        pltpu.sync_copy(ro0.at[pl.ds(0, TAIL // 2), :],
                        tc_hbm.at[pl.ds(NFULL * 128, TAIL // 2), :])


@functools.partial(
    pl.kernel,
    mesh=plsc.VectorSubcoreMesh(core_axis_name="c", subcore_axis_name="s"),
    out_type=jax.ShapeDtypeStruct((S, D, A), jnp.float32),
    compiler_params=pltpu.CompilerParams(
        use_tc_tiling_on_sc=False, needs_layout_passes=False),
    scratch_types=[
        pltpu.VMEM((128,), jnp.int32),
        pltpu.VMEM((128,), jnp.int32),
        pltpu.VMEM((128, D), jnp.float32),
        pltpu.VMEM((128, D), jnp.float32),
        pltpu.VMEM((D, 128), jnp.float32),
        pltpu.VMEM((D, 128), jnp.float32),
        pltpu.SemaphoreType.DMA,
        pltpu.SemaphoreType.DMA,
        pltpu.SemaphoreType.DMA,
        pltpu.SemaphoreType.DMA,
        pltpu.SemaphoreType.DMA,
        pltpu.SemaphoreType.DMA,
    ],
)
def _gather_kernel(xt_hbm, tbl_hbm, out_hbm,
                   idx0, idx1, rows0, rows1, ot0, ot1,
                   g0, g1, w0, w1, i0, i1):
    wid = lax.axis_index("s") * NC + lax.axis_index("c")
    base = wid * UPC
    iota = _iota16()

    def out_slice(k):
        u = base + k
        b = u // (A // 128)
        t = u % (A // 128)
        return out_hbm.at[b, :, pl.ds(t * 128, 128)]

    def idx_desc(k, idxv, sem):
        return pltpu.make_async_copy(
            xt_hbm.at[pl.ds((base + k) * 128, 128)], idxv, sem)

    def fire_gather(idxv, rows, sem):
        pltpu.async_copy(tbl_hbm.at[idxv], rows, sem)

    rbase = [16 * lb + iota for lb in range(8)]

    def step(k, idxv, rows, ot, gsem, wsem, isem):
        pltpu.make_async_copy(tbl_hbm.at[idxv], rows, gsem).wait()

        @pl.when(k + 2 < UPC)
        def _():
            idx_desc(k + 2, idxv, isem).start()

        @pl.when(k >= 2)
        def _():
            pltpu.make_async_copy(ot, out_slice(k - 2), wsem).wait()

        def body(c, carry):
            cv = jnp.full((16,), 0, jnp.int32) + c
            for lb in range(8):
                vals = plsc.load_gather(rows, [rbase[lb], cv])
                ot[c, pl.ds(16 * lb, 16)] = vals
            return carry
        lax.fori_loop(0, D, body, None)

        pltpu.async_copy(ot, out_slice(k), wsem)

        @pl.when(k + 2 < UPC)
        def _():
            idx_desc(k + 2, idxv, isem).wait()
            fire_gather(idxv, rows, gsem)

    idx_desc(0, idx0, i0).start()
    idx_desc(0, idx0, i0).wait()
    fire_gather(idx0, rows0, g0)
    idx_desc(1, idx1, i1).start()
    idx_desc(1, idx1, i1).wait()
    fire_gather(idx1, rows1, g1)

    def pair(g, carry):
        step(2 * g, idx0, rows0, ot0, g0, w0, i0)
        step(2 * g + 1, idx1, rows1, ot1, g1, w1, i1)
        return carry

    lax.fori_loop(0, UPC // 2, pair, None)

    pltpu.make_async_copy(ot0, out_slice(UPC - 2), w0).wait()
    pltpu.make_async_copy(ot1, out_slice(UPC - 1), w1).wait()


def kernel(x, table):
    tail = table[NFULL * SB:].reshape(TAIL // 2, 128)
    table_c = _convert_kernel(table.T, tail)
    xt = x.T.reshape(-1)
    out_p = _gather_kernel(xt, table_c.reshape(VOCAB, D))
    return out_p.transpose(2, 0, 1)


# speedup vs baseline: 1.3795x; 1.3015x over previous
import functools

import jax
import jax.numpy as jnp
from jax import lax
from jax.experimental import pallas as pl
from jax.experimental.pallas import tpu as pltpu
from jax.experimental.pallas import tpu_sc as plsc

VOCAB = 1000000
D = 64
A = 4096
S = 200
B = A * S

_info = plsc.get_sparse_core_info()
NC, NS = _info.num_cores, _info.num_subcores
NW = NC * NS

SB = 256
NFULL = VOCAB // SB
TAIL = VOCAB - NFULL * SB
NPAIR = (NFULL // NW) // 2

UNITS = (S * (A // 128))
UPC = UNITS // NW


def _iota16():
    return lax.iota(jnp.int32, 16)


@functools.partial(
    pl.kernel,
    mesh=plsc.VectorSubcoreMesh(core_axis_name="c", subcore_axis_name="s"),
    out_type=jax.ShapeDtypeStruct((VOCAB // 2, 128), jnp.float32),
    compiler_params=pltpu.CompilerParams(needs_layout_passes=False),
    scratch_types=[
        pltpu.VMEM((64, SB), jnp.float32),
        pltpu.VMEM((64, SB), jnp.float32),
        pltpu.VMEM((128, 128), jnp.float32),
        pltpu.VMEM((128, 128), jnp.float32),
        pltpu.SemaphoreType.DMA,
        pltpu.SemaphoreType.DMA,
        pltpu.SemaphoreType.DMA,
        pltpu.SemaphoreType.DMA,
    ],
)
def _convert_kernel(tt_hbm, tail_hbm, tc_hbm, tin0, tin1, ro0, ro1,
                    r0, r1, w0, w1):
    wid = lax.axis_index("s") * NC + lax.axis_index("c")
    iota = _iota16()

    def fire_reads(u, tin, sem):
        for ct in range(8):
            pltpu.async_copy(
                tt_hbm.at[pl.ds(8 * ct, 8), pl.ds(u * SB, SB)],
                tin.at[pl.ds(8 * ct, 8), :], sem)

    def wait_reads(u, tin, sem):
        for ct in range(8):
            pltpu.make_async_copy(
                tt_hbm.at[pl.ds(8 * ct, 8), pl.ds(u * SB, SB)],
                tin.at[pl.ds(8 * ct, 8), :], sem).wait()

    tbase = [16 * (m % 4) + iota for m in range(8)]

    def transpose(tin, ro, nrow):
        def body(p, carry):
            v0 = jnp.full((16,), 0, jnp.int32) + 2 * p
            v1 = v0 + 1
            vals = [plsc.load_gather(tin, [tbase[m], v0 if m < 4 else v1])
                    for m in range(8)]
            for m in range(8):
                ro[p, pl.ds(16 * m, 16)] = vals[m]
            return carry
        lax.fori_loop(0, nrow, body, None)

    def wdesc(u, ro, wsem):
        return pltpu.make_async_copy(
            ro, tc_hbm.at[pl.ds(u * 128, 128), :], wsem)

    def full_block(g, u, tin, ro, rsem, wsem):
        wait_reads(u, tin, rsem)

        @pl.when(g > 0)
        def _():
            wdesc(u - 2 * NW, ro, wsem).wait()
        transpose(tin, ro, 128)
        pltpu.async_copy(ro, tc_hbm.at[pl.ds(u * 128, 128), :], wsem)

    fire_reads(wid, tin0, r0)

    def pair(g, carry):
        u0 = wid + NW * (2 * g)
        u1 = wid + NW * (2 * g + 1)
        fire_reads(u1, tin1, r1)
        full_block(g, u0, tin0, ro0, r0, w0)

        @pl.when(jnp.logical_or(g < NPAIR - 1, wid < 2))
        def _():
            fire_reads(wid + NW * (2 * g + 2), tin0, r0)
        full_block(g, u1, tin1, ro1, r1, w1)
        return carry

    lax.fori_loop(0, NPAIR, pair, None)

    @pl.when(wid < 2)
    def _():
        u = wid + NW * (2 * NPAIR)
        wait_reads(u, tin0, r0)
        wdesc(u - 2 * NW, ro0, w0).wait()
        transpose(tin0, ro0, 128)
        pltpu.async_copy(ro0, tc_hbm.at[pl.ds(u * 128, 128), :], w0)
        wdesc(u, ro0, w0).wait()

    @pl.when(wid >= 2)
    def _():
        wdesc(wid + NW * (2 * NPAIR - 2), ro0, w0).wait()
    wdesc(wid + NW * (2 * NPAIR - 1), ro1, w1).wait()

    @pl.when(wid == 2)
    def _():
        pltpu.sync_copy(tail_hbm, ro0.at[pl.ds(0, TAIL // 2), :])
        pltpu.sync_copy(ro0.at[pl.ds(0, TAIL // 2), :],
                        tc_hbm.at[pl.ds(NFULL * 128, TAIL // 2), :])


@functools.partial(
    pl.kernel,
    mesh=plsc.VectorSubcoreMesh(core_axis_name="c", subcore_axis_name="s"),
    out_type=jax.ShapeDtypeStruct((S, D, A), jnp.float32),
    compiler_params=pltpu.CompilerParams(
        use_tc_tiling_on_sc=False, needs_layout_passes=False),
    scratch_types=[
        pltpu.VMEM((128,), jnp.int32),
        pltpu.VMEM((128,), jnp.int32),
        pltpu.VMEM((128, D), jnp.float32),
        pltpu.VMEM((128, D), jnp.float32),
        pltpu.VMEM((D, 128), jnp.float32),
        pltpu.VMEM((D, 128), jnp.float32),
        pltpu.SemaphoreType.DMA,
        pltpu.SemaphoreType.DMA,
        pltpu.SemaphoreType.DMA,
        pltpu.SemaphoreType.DMA,
        pltpu.SemaphoreType.DMA,
        pltpu.SemaphoreType.DMA,
    ],
)
def _gather_kernel(xt_hbm, tbl_hbm, out_hbm,
                   idx0, idx1, rows0, rows1, ot0, ot1,
                   g0, g1, w0, w1, i0, i1):
    wid = lax.axis_index("s") * NC + lax.axis_index("c")
    base = wid * UPC
    iota = _iota16()

    def out_slice(k):
        u = base + k
        b = u // (A // 128)
        t = u % (A // 128)
        return out_hbm.at[b, :, pl.ds(t * 128, 128)]

    def idx_desc(k, idxv, sem):
        return pltpu.make_async_copy(
            xt_hbm.at[pl.ds((base + k) * 128, 128)], idxv, sem)

    def fire_gather(idxv, rows, sem):
        pltpu.async_copy(tbl_hbm.at[idxv], rows, sem)

    rbase = [16 * lb + iota for lb in range(8)]

    def step(k, idxv, rows, ot, gsem, wsem, isem):
        pltpu.make_async_copy(tbl_hbm.at[idxv], rows, gsem).wait()

        @pl.when(k + 2 < UPC)
        def _():
            idx_desc(k + 2, idxv, isem).start()

        @pl.when(k >= 2)
        def _():
            pltpu.make_async_copy(ot, out_slice(k - 2), wsem).wait()

        def body(c, carry):
            cv = jnp.full((16,), 0, jnp.int32) + c
            vals = [plsc.load_gather(rows, [rbase[lb], cv]) for lb in range(8)]
            for lb in range(8):
                ot[c, pl.ds(16 * lb, 16)] = vals[lb]
            return carry
        lax.fori_loop(0, D, body, None)

        pltpu.async_copy(ot, out_slice(k), wsem)

        @pl.when(k + 2 < UPC)
        def _():
            idx_desc(k + 2, idxv, isem).wait()
            fire_gather(idxv, rows, gsem)

    idx_desc(0, idx0, i0).start()
    idx_desc(0, idx0, i0).wait()
    fire_gather(idx0, rows0, g0)
    idx_desc(1, idx1, i1).start()
    idx_desc(1, idx1, i1).wait()
    fire_gather(idx1, rows1, g1)

    def pair(g, carry):
        step(2 * g, idx0, rows0, ot0, g0, w0, i0)
        step(2 * g + 1, idx1, rows1, ot1, g1, w1, i1)
        return carry

    lax.fori_loop(0, UPC // 2, pair, None)

    pltpu.make_async_copy(ot0, out_slice(UPC - 2), w0).wait()
    pltpu.make_async_copy(ot1, out_slice(UPC - 1), w1).wait()


def kernel(x, table):
    tail = table[NFULL * SB:].reshape(TAIL // 2, 128)
    table_c = _convert_kernel(table.T, tail)
    xt = x.T.reshape(-1)
    out_p = _gather_kernel(xt, table_c.reshape(VOCAB, D))
    return out_p.transpose(2, 0, 1)


# speedup vs baseline: 1.4362x; 1.0411x over previous
import functools

import jax
import jax.numpy as jnp
from jax import lax
from jax.experimental import pallas as pl
from jax.experimental.pallas import tpu as pltpu
from jax.experimental.pallas import tpu_sc as plsc

VOCAB = 1000000
D = 64
A = 4096
S = 200
B = A * S

_info = plsc.get_sparse_core_info()
NC, NS = _info.num_cores, _info.num_subcores
NW = NC * NS

SB = 256
NFULL = VOCAB // SB
TAIL = VOCAB - NFULL * SB
NPAIR = (NFULL // NW) // 2

UNITS = (S * (A // 128))
UPC = UNITS // NW


def _iota16():
    return lax.iota(jnp.int32, 16)


@functools.partial(
    pl.kernel,
    mesh=plsc.VectorSubcoreMesh(core_axis_name="c", subcore_axis_name="s"),
    out_type=jax.ShapeDtypeStruct((VOCAB // 2, 128), jnp.float32),
    compiler_params=pltpu.CompilerParams(needs_layout_passes=False),
    scratch_types=[
        pltpu.VMEM((64, SB), jnp.float32),
        pltpu.VMEM((64, SB), jnp.float32),
        pltpu.VMEM((128, 128), jnp.float32),
        pltpu.VMEM((128, 128), jnp.float32),
        pltpu.SemaphoreType.DMA,
        pltpu.SemaphoreType.DMA,
        pltpu.SemaphoreType.DMA,
        pltpu.SemaphoreType.DMA,
    ],
)
def _convert_kernel(tt_hbm, tail_hbm, tc_hbm, tin0, tin1, ro0, ro1,
                    r0, r1, w0, w1):
    wid = lax.axis_index("s") * NC + lax.axis_index("c")
    iota = _iota16()

    def fire_reads(u, tin, sem):
        for ct in range(8):
            pltpu.async_copy(
                tt_hbm.at[pl.ds(8 * ct, 8), pl.ds(u * SB, SB)],
                tin.at[pl.ds(8 * ct, 8), :], sem)

    def wait_reads(u, tin, sem):
        for ct in range(8):
            pltpu.make_async_copy(
                tt_hbm.at[pl.ds(8 * ct, 8), pl.ds(u * SB, SB)],
                tin.at[pl.ds(8 * ct, 8), :], sem).wait()

    tbase = [16 * (m % 4) + iota for m in range(8)]

    def transpose(tin, ro, nrow):
        @plsc.parallel_loop(0, nrow, unroll=2)
        def body(p):
            v0 = jnp.full((16,), 0, jnp.int32) + 2 * p
            v1 = v0 + 1
            vals = [plsc.load_gather(tin, [tbase[m], v0 if m < 4 else v1])
                    for m in range(8)]
            for m in range(8):
                ro[p, pl.ds(16 * m, 16)] = vals[m]

    def wdesc(u, ro, wsem):
        return pltpu.make_async_copy(
            ro, tc_hbm.at[pl.ds(u * 128, 128), :], wsem)

    def full_block(g, u, tin, ro, rsem, wsem):
        wait_reads(u, tin, rsem)

        @pl.when(g > 0)
        def _():
            wdesc(u - 2 * NW, ro, wsem).wait()
        transpose(tin, ro, 128)
        pltpu.async_copy(ro, tc_hbm.at[pl.ds(u * 128, 128), :], wsem)

    fire_reads(wid, tin0, r0)

    def pair(g, carry):
        u0 = wid + NW * (2 * g)
        u1 = wid + NW * (2 * g + 1)
        fire_reads(u1, tin1, r1)
        full_block(g, u0, tin0, ro0, r0, w0)

        @pl.when(jnp.logical_or(g < NPAIR - 1, wid < 2))
        def _():
            fire_reads(wid + NW * (2 * g + 2), tin0, r0)
        full_block(g, u1, tin1, ro1, r1, w1)
        return carry

    lax.fori_loop(0, NPAIR, pair, None)

    @pl.when(wid < 2)
    def _():
        u = wid + NW * (2 * NPAIR)
        wait_reads(u, tin0, r0)
        wdesc(u - 2 * NW, ro0, w0).wait()
        transpose(tin0, ro0, 128)
        pltpu.async_copy(ro0, tc_hbm.at[pl.ds(u * 128, 128), :], w0)
        wdesc(u, ro0, w0).wait()

    @pl.when(wid >= 2)
    def _():
        wdesc(wid + NW * (2 * NPAIR - 2), ro0, w0).wait()
    wdesc(wid + NW * (2 * NPAIR - 1), ro1, w1).wait()

    @pl.when(wid == 2)
    def _():
        pltpu.sync_copy(tail_hbm, ro0.at[pl.ds(0, TAIL // 2), :])
        pltpu.sync_copy(ro0.at[pl.ds(0, TAIL // 2), :],
                        tc_hbm.at[pl.ds(NFULL * 128, TAIL // 2), :])


@functools.partial(
    pl.kernel,
    mesh=plsc.VectorSubcoreMesh(core_axis_name="c", subcore_axis_name="s"),
    out_type=jax.ShapeDtypeStruct((S, D, A), jnp.float32),
    compiler_params=pltpu.CompilerParams(
        use_tc_tiling_on_sc=False, needs_layout_passes=False),
    scratch_types=[
        pltpu.VMEM((128,), jnp.int32),
        pltpu.VMEM((128,), jnp.int32),
        pltpu.VMEM((128, D), jnp.float32),
        pltpu.VMEM((128, D), jnp.float32),
        pltpu.VMEM((D, 128), jnp.float32),
        pltpu.VMEM((D, 128), jnp.float32),
        pltpu.SemaphoreType.DMA,
        pltpu.SemaphoreType.DMA,
        pltpu.SemaphoreType.DMA,
        pltpu.SemaphoreType.DMA,
        pltpu.SemaphoreType.DMA,
        pltpu.SemaphoreType.DMA,
    ],
)
def _gather_kernel(xt_hbm, tbl_hbm, out_hbm,
                   idx0, idx1, rows0, rows1, ot0, ot1,
                   g0, g1, w0, w1, i0, i1):
    wid = lax.axis_index("s") * NC + lax.axis_index("c")
    base = wid * UPC
    iota = _iota16()

    def out_slice(k):
        u = base + k
        b = u // (A // 128)
        t = u % (A // 128)
        return out_hbm.at[b, :, pl.ds(t * 128, 128)]

    def idx_desc(k, idxv, sem):
        return pltpu.make_async_copy(
            xt_hbm.at[pl.ds((base + k) * 128, 128)], idxv, sem)

    def fire_gather(idxv, rows, sem):
        pltpu.async_copy(tbl_hbm.at[idxv], rows, sem)

    rbase = [16 * lb + iota for lb in range(8)]

    def step(k, idxv, rows, ot, gsem, wsem, isem):
        pltpu.make_async_copy(tbl_hbm.at[idxv], rows, gsem).wait()

        @pl.when(k + 2 < UPC)
        def _():
            idx_desc(k + 2, idxv, isem).start()

        @pl.when(k >= 2)
        def _():
            pltpu.make_async_copy(ot, out_slice(k - 2), wsem).wait()

        @plsc.parallel_loop(0, D, unroll=2)
        def body(c):
            cv = jnp.full((16,), 0, jnp.int32) + c
            vals = [plsc.load_gather(rows, [rbase[lb], cv]) for lb in range(8)]
            for lb in range(8):
                ot[c, pl.ds(16 * lb, 16)] = vals[lb]

        pltpu.async_copy(ot, out_slice(k), wsem)

        @pl.when(k + 2 < UPC)
        def _():
            idx_desc(k + 2, idxv, isem).wait()
            fire_gather(idxv, rows, gsem)

    idx_desc(0, idx0, i0).start()
    idx_desc(0, idx0, i0).wait()
    fire_gather(idx0, rows0, g0)
    idx_desc(1, idx1, i1).start()
    idx_desc(1, idx1, i1).wait()
    fire_gather(idx1, rows1, g1)

    def pair(g, carry):
        step(2 * g, idx0, rows0, ot0, g0, w0, i0)
        step(2 * g + 1, idx1, rows1, ot1, g1, w1, i1)
        return carry

    lax.fori_loop(0, UPC // 2, pair, None)

    pltpu.make_async_copy(ot0, out_slice(UPC - 2), w0).wait()
    pltpu.make_async_copy(ot1, out_slice(UPC - 1), w1).wait()


def kernel(x, table):
    tail = table[NFULL * SB:].reshape(TAIL // 2, 128)
    table_c = _convert_kernel(table.T, tail)
    xt = x.T.reshape(-1)
    out_p = _gather_kernel(xt, table_c.reshape(VOCAB, D))
    return out_p.transpose(2, 0, 1)


# speedup vs baseline: 4.9592x; 3.4530x over previous
import functools

import jax
import jax.numpy as jnp
from jax import lax
from jax.experimental import pallas as pl
from jax.experimental.pallas import tpu as pltpu
from jax.experimental.pallas import tpu_sc as plsc

VOCAB = 1000000
D = 64
A = 4096
S = 200
B = A * S

_info = plsc.get_sparse_core_info()
NC, NS = _info.num_cores, _info.num_subcores
NW = NC * NS

SB = 256
NFULL = VOCAB // SB
TAIL = VOCAB - NFULL * SB
NPAIR = (NFULL // NW) // 2

UNITS = (S * (A // 128))
UPC = UNITS // NW


def _iota16():
    return lax.iota(jnp.int32, 16)


@functools.partial(
    pl.kernel,
    mesh=plsc.VectorSubcoreMesh(core_axis_name="c", subcore_axis_name="s"),
    out_type=jax.ShapeDtypeStruct((VOCAB // 2, 128), jnp.float32),
    compiler_params=pltpu.CompilerParams(needs_layout_passes=False),
    scratch_types=[
        pltpu.VMEM((64, SB), jnp.float32),
        pltpu.VMEM((64, SB), jnp.float32),
        pltpu.VMEM((128, 128), jnp.float32),
        pltpu.VMEM((128, 128), jnp.float32),
        pltpu.SemaphoreType.DMA,
        pltpu.SemaphoreType.DMA,
        pltpu.SemaphoreType.DMA,
        pltpu.SemaphoreType.DMA,
    ],
)
def _convert_kernel(tt_hbm, tail_hbm, tc_hbm, tin0, tin1, ro0, ro1,
                    r0, r1, w0, w1):
    wid = lax.axis_index("s") * NC + lax.axis_index("c")
    iota = _iota16()

    def fire_reads(u, tin, sem):
        for ct in range(8):
            pltpu.async_copy(
                tt_hbm.at[pl.ds(8 * ct, 8), pl.ds(u * SB, SB)],
                tin.at[pl.ds(8 * ct, 8), :], sem)

    def wait_reads(u, tin, sem):
        for ct in range(8):
            pltpu.make_async_copy(
                tt_hbm.at[pl.ds(8 * ct, 8), pl.ds(u * SB, SB)],
                tin.at[pl.ds(8 * ct, 8), :], sem).wait()

    rotv = [(iota + d) & 15 for d in range(16)]

    def transpose(tin, ro):
        @plsc.parallel_loop(0, 64)
        def body(i):
            l0 = (i // 4) * 16
            c0 = (i % 4) * 16
            cvec = c0 + iota
            for d in range(16):
                lvec = l0 + rotv[d]
                vals = plsc.load_gather(tin, [cvec, lvec])
                pvec = lvec >> 1
                qvec = ((lvec & 1) << 6) + cvec
                plsc.store_scatter(ro, [pvec, qvec], vals)

    def wdesc(u, ro, wsem):
        return pltpu.make_async_copy(
            ro, tc_hbm.at[pl.ds(u * 128, 128), :], wsem)

    def full_block(g, u, tin, ro, rsem, wsem):
        wait_reads(u, tin, rsem)

        @pl.when(g > 0)
        def _():
            wdesc(u - 2 * NW, ro, wsem).wait()
        transpose(tin, ro)
        pltpu.async_copy(ro, tc_hbm.at[pl.ds(u * 128, 128), :], wsem)

    fire_reads(wid, tin0, r0)

    def pair(g, carry):
        u0 = wid + NW * (2 * g)
        u1 = wid + NW * (2 * g + 1)
        fire_reads(u1, tin1, r1)
        full_block(g, u0, tin0, ro0, r0, w0)

        @pl.when(jnp.logical_or(g < NPAIR - 1, wid < 2))
        def _():
            fire_reads(wid + NW * (2 * g + 2), tin0, r0)
        full_block(g, u1, tin1, ro1, r1, w1)
        return carry

    lax.fori_loop(0, NPAIR, pair, None)

    @pl.when(wid < 2)
    def _():
        u = wid + NW * (2 * NPAIR)
        wait_reads(u, tin0, r0)
        wdesc(u - 2 * NW, ro0, w0).wait()
        transpose(tin0, ro0)
        pltpu.async_copy(ro0, tc_hbm.at[pl.ds(u * 128, 128), :], w0)
        wdesc(u, ro0, w0).wait()

    @pl.when(wid >= 2)
    def _():
        wdesc(wid + NW * (2 * NPAIR - 2), ro0, w0).wait()
    wdesc(wid + NW * (2 * NPAIR - 1), ro1, w1).wait()

    @pl.when(wid == 2)
    def _():
        pltpu.sync_copy(tail_hbm, ro0.at[pl.ds(0, TAIL // 2), :])
        pltpu.sync_copy(ro0.at[pl.ds(0, TAIL // 2), :],
                        tc_hbm.at[pl.ds(NFULL * 128, TAIL // 2), :])


@functools.partial(
    pl.kernel,
    mesh=plsc.VectorSubcoreMesh(core_axis_name="c", subcore_axis_name="s"),
    out_type=jax.ShapeDtypeStruct((S, D, A), jnp.float32),
    compiler_params=pltpu.CompilerParams(
        use_tc_tiling_on_sc=False, needs_layout_passes=False),
    scratch_types=[
        pltpu.VMEM((128,), jnp.int32),
        pltpu.VMEM((128,), jnp.int32),
        pltpu.VMEM((128, D), jnp.float32),
        pltpu.VMEM((128, D), jnp.float32),
        pltpu.VMEM((D, 128), jnp.float32),
        pltpu.VMEM((D, 128), jnp.float32),
        pltpu.SemaphoreType.DMA,
        pltpu.SemaphoreType.DMA,
        pltpu.SemaphoreType.DMA,
        pltpu.SemaphoreType.DMA,
        pltpu.SemaphoreType.DMA,
        pltpu.SemaphoreType.DMA,
    ],
)
def _gather_kernel(xt_hbm, tbl_hbm, out_hbm,
                   idx0, idx1, rows0, rows1, ot0, ot1,
                   g0, g1, w0, w1, i0, i1):
    wid = lax.axis_index("s") * NC + lax.axis_index("c")
    base = wid * UPC
    iota = _iota16()

    def out_slice(k):
        u = base + k
        b = u // (A // 128)
        t = u % (A // 128)
        return out_hbm.at[b, :, pl.ds(t * 128, 128)]

    def idx_desc(k, idxv, sem):
        return pltpu.make_async_copy(
            xt_hbm.at[pl.ds((base + k) * 128, 128)], idxv, sem)

    def fire_gather(idxv, rows, sem):
        pltpu.async_copy(tbl_hbm.at[idxv], rows, sem)

    rotv = [(iota + d) & 15 for d in range(16)]

    def step(k, idxv, rows, ot, gsem, wsem, isem):
        pltpu.make_async_copy(tbl_hbm.at[idxv], rows, gsem).wait()

        @pl.when(k + 2 < UPC)
        def _():
            idx_desc(k + 2, idxv, isem).start()

        @pl.when(k >= 2)
        def _():
            pltpu.make_async_copy(ot, out_slice(k - 2), wsem).wait()

        @plsc.parallel_loop(0, 32)
        def body(i):
            c0 = (i // 8) * 16
            l0 = (i % 8) * 16
            cvec = c0 + iota
            for d in range(16):
                lvec = l0 + rotv[d]
                vals = plsc.load_gather(rows, [lvec, cvec])
                plsc.store_scatter(ot, [cvec, lvec], vals)

        pltpu.async_copy(ot, out_slice(k), wsem)

        @pl.when(k + 2 < UPC)
        def _():
            idx_desc(k + 2, idxv, isem).wait()
            fire_gather(idxv, rows, gsem)

    idx_desc(0, idx0, i0).start()
    idx_desc(0, idx0, i0).wait()
    fire_gather(idx0, rows0, g0)
    idx_desc(1, idx1, i1).start()
    idx_desc(1, idx1, i1).wait()
    fire_gather(idx1, rows1, g1)

    def pair(g, carry):
        step(2 * g, idx0, rows0, ot0, g0, w0, i0)
        step(2 * g + 1, idx1, rows1, ot1, g1, w1, i1)
        return carry

    lax.fori_loop(0, UPC // 2, pair, None)

    pltpu.make_async_copy(ot0, out_slice(UPC - 2), w0).wait()
    pltpu.make_async_copy(ot1, out_slice(UPC - 1), w1).wait()


def kernel(x, table):
    tail = table[NFULL * SB:].reshape(TAIL // 2, 128)
    table_c = _convert_kernel(table.T, tail)
    xt = x.T.reshape(-1)
    out_p = _gather_kernel(xt, table_c.reshape(VOCAB, D))
    return out_p.transpose(2, 0, 1)


# speedup vs baseline: 5.1080x; 1.0300x over previous
import functools

import jax
import jax.numpy as jnp
from jax import lax
from jax.experimental import pallas as pl
from jax.experimental.pallas import tpu as pltpu
from jax.experimental.pallas import tpu_sc as plsc

VOCAB = 1000000
D = 64
A = 4096
S = 200
B = A * S

_info = plsc.get_sparse_core_info()
NC, NS = _info.num_cores, _info.num_subcores
NW = NC * NS

SB = 256
NFULL = VOCAB // SB
TAIL = VOCAB - NFULL * SB
NPAIR = (NFULL // NW) // 2

UNITS = (S * (A // 128))
UPC = UNITS // NW


def _iota16():
    return lax.iota(jnp.int32, 16)


@functools.partial(
    pl.kernel,
    mesh=plsc.VectorSubcoreMesh(core_axis_name="c", subcore_axis_name="s"),
    out_type=jax.ShapeDtypeStruct((VOCAB // 2, 128), jnp.float32),
    compiler_params=pltpu.CompilerParams(needs_layout_passes=False),
    scratch_types=[
        pltpu.VMEM((64, SB), jnp.float32),
        pltpu.VMEM((64, SB), jnp.float32),
        pltpu.VMEM((128, 128), jnp.float32),
        pltpu.VMEM((128, 128), jnp.float32),
        pltpu.SemaphoreType.DMA,
        pltpu.SemaphoreType.DMA,
        pltpu.SemaphoreType.DMA,
        pltpu.SemaphoreType.DMA,
    ],
)
def _convert_kernel(tt_hbm, tail_hbm, tc_hbm, tin0, tin1, ro0, ro1,
                    r0, r1, w0, w1):
    wid = lax.axis_index("s") * NC + lax.axis_index("c")
    iota = _iota16()

    def fire_reads(u, tin, sem):
        for ct in range(8):
            pltpu.async_copy(
                tt_hbm.at[pl.ds(8 * ct, 8), pl.ds(u * SB, SB)],
                tin.at[pl.ds(8 * ct, 8), :], sem)

    def wait_reads(u, tin, sem):
        for ct in range(8):
            pltpu.make_async_copy(
                tt_hbm.at[pl.ds(8 * ct, 8), pl.ds(u * SB, SB)],
                tin.at[pl.ds(8 * ct, 8), :], sem).wait()

    rotv = [(iota + d) & 15 for d in range(16)]

    def transpose(tin, ro):
        @plsc.parallel_loop(0, 64, unroll=2)
        def body(i):
            l0 = (i // 4) * 16
            c0 = (i % 4) * 16
            cvec = c0 + iota
            for d in range(16):
                lvec = l0 + rotv[d]
                vals = plsc.load_gather(tin, [cvec, lvec])
                pvec = lvec >> 1
                qvec = ((lvec & 1) << 6) + cvec
                plsc.store_scatter(ro, [pvec, qvec], vals)

    def wdesc(u, ro, wsem):
        return pltpu.make_async_copy(
            ro, tc_hbm.at[pl.ds(u * 128, 128), :], wsem)

    def full_block(g, u, tin, ro, rsem, wsem):
        wait_reads(u, tin, rsem)

        @pl.when(g > 0)
        def _():
            wdesc(u - 2 * NW, ro, wsem).wait()
        transpose(tin, ro)
        pltpu.async_copy(ro, tc_hbm.at[pl.ds(u * 128, 128), :], wsem)

    fire_reads(wid, tin0, r0)

    def pair(g, carry):
        u0 = wid + NW * (2 * g)
        u1 = wid + NW * (2 * g + 1)
        fire_reads(u1, tin1, r1)
        full_block(g, u0, tin0, ro0, r0, w0)

        @pl.when(jnp.logical_or(g < NPAIR - 1, wid < 2))
        def _():
            fire_reads(wid + NW * (2 * g + 2), tin0, r0)
        full_block(g, u1, tin1, ro1, r1, w1)
        return carry

    lax.fori_loop(0, NPAIR, pair, None)

    @pl.when(wid < 2)
    def _():
        u = wid + NW * (2 * NPAIR)
        wait_reads(u, tin0, r0)
        wdesc(u - 2 * NW, ro0, w0).wait()
        transpose(tin0, ro0)
        pltpu.async_copy(ro0, tc_hbm.at[pl.ds(u * 128, 128), :], w0)
        wdesc(u, ro0, w0).wait()

    @pl.when(wid >= 2)
    def _():
        wdesc(wid + NW * (2 * NPAIR - 2), ro0, w0).wait()
    wdesc(wid + NW * (2 * NPAIR - 1), ro1, w1).wait()

    @pl.when(wid == 2)
    def _():
        pltpu.sync_copy(tail_hbm, ro0.at[pl.ds(0, TAIL // 2), :])
        pltpu.sync_copy(ro0.at[pl.ds(0, TAIL // 2), :],
                        tc_hbm.at[pl.ds(NFULL * 128, TAIL // 2), :])


@functools.partial(
    pl.kernel,
    mesh=plsc.VectorSubcoreMesh(core_axis_name="c", subcore_axis_name="s"),
    out_type=jax.ShapeDtypeStruct((S, D, A), jnp.float32),
    compiler_params=pltpu.CompilerParams(
        use_tc_tiling_on_sc=False, needs_layout_passes=False),
    scratch_types=[
        pltpu.VMEM((128,), jnp.int32),
        pltpu.VMEM((128,), jnp.int32),
        pltpu.VMEM((128, D), jnp.float32),
        pltpu.VMEM((128, D), jnp.float32),
        pltpu.VMEM((D, 128), jnp.float32),
        pltpu.VMEM((D, 128), jnp.float32),
        pltpu.SemaphoreType.DMA,
        pltpu.SemaphoreType.DMA,
        pltpu.SemaphoreType.DMA,
        pltpu.SemaphoreType.DMA,
        pltpu.SemaphoreType.DMA,
        pltpu.SemaphoreType.DMA,
    ],
)
def _gather_kernel(xt_hbm, tbl_hbm, out_hbm,
                   idx0, idx1, rows0, rows1, ot0, ot1,
                   g0, g1, w0, w1, i0, i1):
    wid = lax.axis_index("s") * NC + lax.axis_index("c")
    base = wid * UPC
    iota = _iota16()

    def out_slice(k):
        u = base + k
        b = u // (A // 128)
        t = u % (A // 128)
        return out_hbm.at[b, :, pl.ds(t * 128, 128)]

    def idx_desc(k, idxv, sem):
        return pltpu.make_async_copy(
            xt_hbm.at[pl.ds((base + k) * 128, 128)], idxv, sem)

    def fire_gather(idxv, rows, sem):
        pltpu.async_copy(tbl_hbm.at[idxv], rows, sem)

    rotv = [(iota + d) & 15 for d in range(16)]

    def step(k, idxv, rows, ot, gsem, wsem, isem):
        pltpu.make_async_copy(tbl_hbm.at[idxv], rows, gsem).wait()

        @pl.when(k + 2 < UPC)
        def _():
            idx_desc(k + 2, idxv, isem).start()

        @pl.when(k >= 2)
        def _():
            pltpu.make_async_copy(ot, out_slice(k - 2), wsem).wait()

        @plsc.parallel_loop(0, 32, unroll=2)
        def body(i):
            c0 = (i // 8) * 16
            l0 = (i % 8) * 16
            cvec = c0 + iota
            for d in range(16):
                lvec = l0 + rotv[d]
                vals = plsc.load_gather(rows, [lvec, cvec])
                plsc.store_scatter(ot, [cvec, lvec], vals)

        pltpu.async_copy(ot, out_slice(k), wsem)

        @pl.when(k + 2 < UPC)
        def _():
            idx_desc(k + 2, idxv, isem).wait()
            fire_gather(idxv, rows, gsem)

    idx_desc(0, idx0, i0).start()
    idx_desc(0, idx0, i0).wait()
    fire_gather(idx0, rows0, g0)
    idx_desc(1, idx1, i1).start()
    idx_desc(1, idx1, i1).wait()
    fire_gather(idx1, rows1, g1)

    def pair(g, carry):
        step(2 * g, idx0, rows0, ot0, g0, w0, i0)
        step(2 * g + 1, idx1, rows1, ot1, g1, w1, i1)
        return carry

    lax.fori_loop(0, UPC // 2, pair, None)

    pltpu.make_async_copy(ot0, out_slice(UPC - 2), w0).wait()
    pltpu.make_async_copy(ot1, out_slice(UPC - 1), w1).wait()


def kernel(x, table):
    tail = table[NFULL * SB:].reshape(TAIL // 2, 128)
    table_c = _convert_kernel(table.T, tail)
    xt = x.T.reshape(-1)
    out_p = _gather_kernel(xt, table_c.reshape(VOCAB, D))
    return out_p.transpose(2, 0, 1)
